# speedup vs baseline: 18.1503x; 18.1503x over previous
import jax
import jax.numpy as jnp
from jax import lax
from jax.experimental import pallas as pl

B = 20
P = 1000
K = 15
KS = 5
FN = 10
M = P // 5
NA = 4

_BIG = 1e9
_interp = False


def _topk_idx(cur, il, n):
    idxs = []
    for _ in range(K):
        mval = jnp.min(cur, axis=1, keepdims=True)
        eq = cur == mval
        idx = jnp.min(jnp.where(eq, il, n), axis=1, keepdims=True)
        idxs.append(idx)
        cur = cur + jnp.where(il == idx, _BIG, 0.0)
    return idxs


def _stage_a(pcol_ref, prow_ref, wm_ref, w1_ref, b1_ref, w2_ref, b2_ref,
             fdd_ref):
    f32 = jnp.float32
    pc = pcol_ref[0]
    pr = prow_ref[0]
    il = lax.broadcasted_iota(jnp.int32, (P, P), 1)
    isub = lax.broadcasted_iota(jnp.int32, (P, P), 0)
    d2 = ((pc[:, 0:1] - pr[0:1, :]) ** 2
          + (pc[:, 1:2] - pr[1:2, :]) ** 2
          + (pc[:, 2:3] - pr[2:3, :]) ** 2)
    cur = d2 + jnp.where(il == isub, _BIG, 0.0)
    idxs = _topk_idx(cur, il, P)

    l15 = lax.broadcasted_iota(jnp.int32, (1, 3 * KS), 1)
    gridf = (l15 % KS).astype(f32)
    s3 = lax.broadcasted_iota(jnp.int32, (3, 3 * KS), 0)
    rsel = (l15 // KS == s3).astype(f32)
    facc = jnp.zeros((P, 3 * FN), f32)
    for k in range(K):
        m = (il == idxs[k]).astype(f32)
        g = jnp.dot(m, pc)
        rel = g - pc
        u = (jnp.tanh(rel) + 1.0) * (0.5 * (KS - 1))
        urep = jnp.dot(u, rsel)
        basis = jnp.maximum(0.0, 1.0 - jnp.abs(urep - gridf))
        facc = facc + jnp.dot(basis, wm_ref[...])
    f3d = jax.nn.sigmoid(facc * (1.0 / K))

    gsrc = jnp.concatenate([f3d, pc], axis=1)
    shift = jnp.concatenate([jnp.zeros((P, 3 * FN), f32), pc], axis=1)
    w1 = w1_ref[...]
    w2 = w2_ref[...]
    b1 = b1_ref[...]
    b2 = b2_ref[...]
    fddm = jnp.full((P, 20), -jnp.inf, f32)
    for k in range(K):
        m = (il == idxs[k]).astype(f32)
        nf = jnp.dot(m, gsrc) - shift
        h = jnp.maximum(jnp.dot(nf, w1) + b1, 0.0)
        h2 = jnp.maximum(jnp.dot(h, w2) + b2, 0.0)
        fddm = jnp.maximum(fddm, h2)
    fdd_ref[0] = jax.nn.sigmoid(fddm)


def _fps(prow_ref, out_ref):
    f32 = jnp.float32
    px = prow_ref[0]
    py = prow_ref[1]
    pz = prow_ref[2]
    lane = lax.broadcasted_iota(jnp.int32, (B, P), 1)
    lane2 = lax.broadcasted_iota(jnp.int32, (B, M), 1)
    cx0 = px[:, 0:1]
    cy0 = py[:, 0:1]
    cz0 = pz[:, 0:1]
    mind = (px - cx0) ** 2 + (py - cy0) ** 2 + (pz - cz0) ** 2
    p2x = jnp.where(lane2 == 0, cx0, 0.0)
    p2y = jnp.where(lane2 == 0, cy0, 0.0)
    p2z = jnp.where(lane2 == 0, cz0, 0.0)

    def body(i, st):
        mind, p2x, p2y, p2z = st
        mx = jnp.max(mind, axis=1, keepdims=True)
        eq = mind == mx
        nxt = jnp.min(jnp.where(eq, lane, P), axis=1, keepdims=True)
        oh = (lane == nxt).astype(f32)
        cx = jnp.sum(px * oh, axis=1, keepdims=True)
        cy = jnp.sum(py * oh, axis=1, keepdims=True)
        cz = jnp.sum(pz * oh, axis=1, keepdims=True)
        d = (px - cx) ** 2 + (py - cy) ** 2 + (pz - cz) ** 2
        mind = jnp.minimum(mind, d)
        sel = lane2 == i
        p2x = jnp.where(sel, cx, p2x)
        p2y = jnp.where(sel, cy, p2y)
        p2z = jnp.where(sel, cz, p2z)
        return (mind, p2x, p2y, p2z)

    _, p2x, p2y, p2z = lax.fori_loop(1, M, body, (mind, p2x, p2y, p2z))
    out_ref[0] = p2x
    out_ref[1] = p2y
    out_ref[2] = p2z


def _stage_c(p2c_ref, p2r_ref, fdd_ref, w3_ref, b3_ref, w4_ref, b4_ref,
             ys_ref):
    f32 = jnp.float32
    b = pl.program_id(0)
    pc = p2c_ref[0]
    pr = p2r_ref[0]
    il = lax.broadcasted_iota(jnp.int32, (M, M), 1)
    isub = lax.broadcasted_iota(jnp.int32, (M, M), 0)
    d2 = ((pc[:, 0:1] - pr[0:1, :]) ** 2
          + (pc[:, 1:2] - pr[1:2, :]) ** 2
          + (pc[:, 2:3] - pr[2:3, :]) ** 2)
    cur = d2 + jnp.where(il == isub, _BIG, 0.0)
    idxs = _topk_idx(cur, il, M)

    sub = fdd_ref[pl.ds(b * M, M), :]
    macc = jnp.zeros((M, 20), f32)
    mmax = jnp.full((M, 20), -jnp.inf, f32)
    for k in range(K):
        m = (il == idxs[k]).astype(f32)
        g = jnp.dot(m, sub)
        macc = macc + g
        mmax = jnp.maximum(mmax, g)
    x2 = jnp.concatenate([sub, macc * (1.0 / K), mmax], axis=1)
    h = jnp.maximum(jnp.dot(x2, w3_ref[...]) + b3_ref[...], 0.0)
    h2 = jnp.maximum(jnp.dot(h, w4_ref[...]) + b4_ref[...], 0.0)
    f2 = jax.nn.sigmoid(h2)
    ys_ref[0] = jnp.mean(f2, axis=0, keepdims=True)


def _head(ys_ref, wn1_ref, bn1_ref, wn2_ref, bn2_ref, out_ref):
    ys = ys_ref[...]
    y1 = jnp.dot(ys, wn1_ref[...]) + bn1_ref[...]
    y1 = jnp.where(y1 > 0, y1, jnp.exp(jnp.minimum(y1, 0.0)) - 1.0)
    z = jnp.dot(y1, wn2_ref[...]) + bn2_ref[...]
    s = z - jnp.max(z, axis=1, keepdims=True)
    out_ref[...] = s - jnp.log(jnp.sum(jnp.exp(s), axis=1, keepdims=True))


def kernel(pos, edge_index, batch, Wsp, W1, b1, W2, b2, W3, b3, W4, b4,
           Wn1, bn1, Wn2, bn2):
    del edge_index, batch
    f32 = jnp.float32
    pos = pos.astype(f32)
    posg = pos.reshape(B, P, 3)
    pg4 = posg[:NA]
    pg4_row = jnp.transpose(pg4, (0, 2, 1))
    pall_row = jnp.transpose(posg, (2, 0, 1))

    w_cjf = jnp.transpose(Wsp.astype(f32), (2, 1, 0))
    wmat = (w_cjf[:, :, :, None]
            * jnp.eye(3, dtype=f32)[:, None, None, :]).reshape(3 * KS, 3 * FN)

    fdd = pl.pallas_call(
        _stage_a,
        grid=(NA,),
        in_specs=[
            pl.BlockSpec((1, P, 3), lambda g: (g, 0, 0)),
            pl.BlockSpec((1, 3, P), lambda g: (g, 0, 0)),
            pl.BlockSpec((3 * KS, 3 * FN), lambda g: (0, 0)),
            pl.BlockSpec((3 * FN + 3, 20), lambda g: (0, 0)),
            pl.BlockSpec((1, 20), lambda g: (0, 0)),
            pl.BlockSpec((20, 20), lambda g: (0, 0)),
            pl.BlockSpec((1, 20), lambda g: (0, 0)),
        ],
        out_specs=pl.BlockSpec((1, P, 20), lambda g: (g, 0, 0)),
        out_shape=jax.ShapeDtypeStruct((NA, P, 20), f32),
        interpret=_interp,
    )(pg4, pg4_row, wmat, W1, b1.reshape(1, 20), W2, b2.reshape(1, 20))
    fdd_all = fdd.reshape(NA * P, 20)

    p2 = pl.pallas_call(
        _fps,
        grid=(1,),
        in_specs=[pl.BlockSpec((3, B, P), lambda i: (0, 0, 0))],
        out_specs=pl.BlockSpec((3, B, M), lambda i: (0, 0, 0)),
        out_shape=jax.ShapeDtypeStruct((3, B, M), f32),
        interpret=_interp,
    )(pall_row)
    p2col = jnp.transpose(p2, (1, 2, 0))
    p2row = jnp.transpose(p2, (1, 0, 2))

    ys = pl.pallas_call(
        _stage_c,
        grid=(B,),
        in_specs=[
            pl.BlockSpec((1, M, 3), lambda b: (b, 0, 0)),
            pl.BlockSpec((1, 3, M), lambda b: (b, 0, 0)),
            pl.BlockSpec((NA * P, 20), lambda b: (0, 0)),
            pl.BlockSpec((60, 64), lambda b: (0, 0)),
            pl.BlockSpec((1, 64), lambda b: (0, 0)),
            pl.BlockSpec((64, 32), lambda b: (0, 0)),
            pl.BlockSpec((1, 32), lambda b: (0, 0)),
        ],
        out_specs=pl.BlockSpec((1, 1, 32), lambda b: (b, 0, 0)),
        out_shape=jax.ShapeDtypeStruct((B, 1, 32), f32),
        interpret=_interp,
    )(p2col, p2row, fdd_all, W3, b3.reshape(1, 64), W4, b4.reshape(1, 32))

    out = pl.pallas_call(
        _head,
        in_specs=[
            pl.BlockSpec((B, 32), lambda: (0, 0)),
            pl.BlockSpec((32, 256), lambda: (0, 0)),
            pl.BlockSpec((1, 256), lambda: (0, 0)),
            pl.BlockSpec((256, 40), lambda: (0, 0)),
            pl.BlockSpec((1, 40), lambda: (0, 0)),
        ],
        out_specs=pl.BlockSpec((B, 40), lambda: (0, 0)),
        out_shape=jax.ShapeDtypeStruct((B, 40), f32),
        interpret=_interp,
    )(ys.reshape(B, 32), Wn1, bn1.reshape(1, 256), Wn2, bn2.reshape(1, 40))
    return out
